# baseline (speedup 1.0000x reference)
import jax
import jax.numpy as jnp
from jax.experimental import pallas as pl
from jax.experimental.pallas import tpu as pltpu

_NUM_EMBED = 1024
_EMBED_DIM = 64
_BETA = 0.25
_CHUNK = 4096


def _vq_body(x_ref, emb_ref, q_ref, idx_ref, cnt_ref, out_ref):
    b = pl.program_id(0)
    t = pl.program_id(1)
    nb = pl.num_programs(0)
    nt = pl.num_programs(1)
    xb = x_ref[0]
    emb = emb_ref[...]

    e_sq = jnp.sum(emb * emb, axis=1, keepdims=True)
    prod = jax.lax.dot_general(
        emb, xb, (((1,), (0,)), ((), ())),
        preferred_element_type=jnp.float32)
    d = e_sq - 2.0 * prod

    min_d = jnp.min(d, axis=0, keepdims=True)
    iota0 = jax.lax.broadcasted_iota(jnp.int32, d.shape, 0)
    idx = jnp.min(jnp.where(d <= min_d, iota0, jnp.int32(_NUM_EMBED)),
                  axis=0)
    onehot = (iota0 == idx[None, :]).astype(jnp.float32)

    qb = jax.lax.dot_general(
        emb, onehot, (((0,), (0,)), ((), ())),
        preferred_element_type=jnp.float32)

    q_ref[0] = qb
    idx_ref[0, 0, 0] = idx

    x_sq = jnp.sum(xb * xb, axis=0)
    loss_s = jnp.sum(min_d[0] + x_sq)
    cnt_mat = jax.lax.dot_general(
        jnp.ones((8, _CHUNK), jnp.bfloat16), onehot.astype(jnp.bfloat16),
        (((1,), (1,)), ((), ())),
        preferred_element_type=jnp.float32)
    cnt_part = cnt_mat[0:1, :]

    first = jnp.logical_and(b == 0, t == 0)
    last = jnp.logical_and(b == nb - 1, t == nt - 1)

    @pl.when(first)
    def _init():
        cnt_ref[...] = cnt_part
        out_ref[...] = jnp.full(out_ref.shape, loss_s, jnp.float32)

    @pl.when(jnp.logical_not(first))
    def _acc():
        cnt_ref[...] += cnt_part
        out_ref[...] += loss_s

    @pl.when(last)
    def _finalize():
        n_tokens = jnp.float32(nb * nt * _CHUNK)
        loss_row = out_ref[...] * (_BETA / (n_tokens * _EMBED_DIM))
        p = cnt_ref[...] / n_tokens
        ent = jnp.sum(p * jnp.log(p + 1e-10), keepdims=True)
        perp_row = jnp.broadcast_to(jnp.exp(-ent), out_ref.shape)
        lane = jax.lax.broadcasted_iota(jnp.int32, out_ref.shape, 1)
        out_ref[...] = jnp.where(lane == 0, loss_row, perp_row)


def kernel(inputs, emb_weight):
    B, C, H, W = inputs.shape
    HW = H * W
    n_chunks = HW // _CHUNK
    x3 = inputs.reshape(B, C, HW)

    grid = (B, n_chunks)
    q3, idx3, _counts, scalars = pl.pallas_call(
        _vq_body,
        grid=grid,
        in_specs=[
            pl.BlockSpec((1, C, _CHUNK), lambda b, t: (b, 0, t)),
            pl.BlockSpec((_NUM_EMBED, _EMBED_DIM), lambda b, t: (0, 0)),
        ],
        out_specs=[
            pl.BlockSpec((1, C, _CHUNK), lambda b, t: (b, 0, t)),
            pl.BlockSpec((1, 1, 1, _CHUNK), lambda b, t: (b, t, 0, 0)),
            pl.BlockSpec((1, _NUM_EMBED), lambda b, t: (0, 0)),
            pl.BlockSpec((1, 128), lambda b, t: (0, 0)),
        ],
        out_shape=[
            jax.ShapeDtypeStruct((B, C, HW), jnp.float32),
            jax.ShapeDtypeStruct((B, n_chunks, 1, _CHUNK), jnp.int32),
            jax.ShapeDtypeStruct((1, _NUM_EMBED), jnp.float32),
            jax.ShapeDtypeStruct((1, 128), jnp.float32),
        ],
        compiler_params=pltpu.CompilerParams(
            dimension_semantics=("arbitrary", "arbitrary")),
    )(x3, emb_weight)

    loss = scalars[0, 0]
    perplexity = scalars[0, 1]
    q_out = q3.reshape(B, C, H, W)
    encoding_indices = idx3.reshape(B, H, W)
    return loss, q_out, perplexity, encoding_indices


# speedup vs baseline: 1.1242x; 1.1242x over previous
import jax
import jax.numpy as jnp
from jax.experimental import pallas as pl
from jax.experimental.pallas import tpu as pltpu

_NUM_EMBED = 1024
_EMBED_DIM = 64
_BETA = 0.25
_CHUNK = 4096


def _vq_body(x_ref, emb_ref, q_ref, idx_ref, cnt_ref, out_ref):
    b = pl.program_id(0)
    t = pl.program_id(1)
    nb = pl.num_programs(0)
    nt = pl.num_programs(1)
    xb = x_ref[0]
    emb = emb_ref[...]

    e_sq = jnp.sum(emb * emb, axis=1, keepdims=True)
    prod = jax.lax.dot_general(
        emb, xb, (((1,), (0,)), ((), ())),
        preferred_element_type=jnp.float32)
    d = e_sq - 2.0 * prod

    min_d = jnp.min(d, axis=0, keepdims=True)
    maskf = (d <= min_d).astype(jnp.float32)
    mask_bf = maskf.astype(jnp.bfloat16)

    rid = jax.lax.broadcasted_iota(jnp.int32, (8, _NUM_EMBED), 0)
    cid = jax.lax.broadcasted_iota(jnp.int32, (8, _NUM_EMBED), 1)
    aux = jnp.where(
        rid == 0, 1,
        jnp.where(rid == 1, cid // 4,
                  jnp.where(rid == 2, cid % 4, 0))).astype(jnp.bfloat16)
    statm = jax.lax.dot_general(
        aux, mask_bf, (((1,), (0,)), ((), ())),
        preferred_element_type=jnp.float32)
    idx = (4.0 * statm[1] + statm[2] + 0.5).astype(jnp.int32)

    qb = jax.lax.dot_general(
        emb, maskf, (((0,), (0,)), ((), ())),
        preferred_element_type=jnp.float32)

    cnt_mat = jax.lax.dot_general(
        jnp.ones((8, _CHUNK), jnp.bfloat16), mask_bf,
        (((1,), (1,)), ((), ())),
        preferred_element_type=jnp.float32)
    cnt_part = cnt_mat[0:1, :]

    q_ref[0] = qb
    idx_ref[0, 0, 0] = idx

    x_sq = jnp.sum(xb * xb, axis=0)
    loss_s = jnp.sum(min_d[0] + x_sq)

    first = jnp.logical_and(b == 0, t == 0)
    last = jnp.logical_and(b == nb - 1, t == nt - 1)

    @pl.when(first)
    def _init():
        cnt_ref[...] = cnt_part
        out_ref[...] = jnp.full(out_ref.shape, loss_s, jnp.float32)

    @pl.when(jnp.logical_not(first))
    def _acc():
        cnt_ref[...] += cnt_part
        out_ref[...] += loss_s

    n_sel = jnp.sum(statm[0])
    @pl.when(n_sel > _CHUNK + 0.5)
    def _fix_ties():
        iota0 = jax.lax.broadcasted_iota(jnp.int32, d.shape, 0)
        idx_x = jnp.min(jnp.where(d <= min_d, iota0, jnp.int32(_NUM_EMBED)),
                        axis=0)
        onehot = (iota0 == idx_x[None, :]).astype(jnp.float32)
        q_ref[0] = jax.lax.dot_general(
            emb, onehot, (((0,), (0,)), ((), ())),
            preferred_element_type=jnp.float32)
        idx_ref[0, 0, 0] = idx_x
        cmx = jax.lax.dot_general(
            jnp.ones((8, _CHUNK), jnp.bfloat16), onehot.astype(jnp.bfloat16),
            (((1,), (1,)), ((), ())),
            preferred_element_type=jnp.float32)
        cnt_ref[...] += cmx[0:1, :] - cnt_part

    @pl.when(last)
    def _finalize():
        n_tokens = jnp.float32(nb * nt * _CHUNK)
        loss_row = out_ref[...] * (_BETA / (n_tokens * _EMBED_DIM))
        p = cnt_ref[...] / n_tokens
        ent = jnp.sum(p * jnp.log(p + 1e-10), keepdims=True)
        perp_row = jnp.broadcast_to(jnp.exp(-ent), out_ref.shape)
        lane = jax.lax.broadcasted_iota(jnp.int32, out_ref.shape, 1)
        out_ref[...] = jnp.where(lane == 0, loss_row, perp_row)


def kernel(inputs, emb_weight):
    B, C, H, W = inputs.shape
    HW = H * W
    n_chunks = HW // _CHUNK
    x3 = inputs.reshape(B, C, HW)

    grid = (B, n_chunks)
    q3, idx3, _counts, scalars = pl.pallas_call(
        _vq_body,
        grid=grid,
        in_specs=[
            pl.BlockSpec((1, C, _CHUNK), lambda b, t: (b, 0, t)),
            pl.BlockSpec((_NUM_EMBED, _EMBED_DIM), lambda b, t: (0, 0)),
        ],
        out_specs=[
            pl.BlockSpec((1, C, _CHUNK), lambda b, t: (b, 0, t)),
            pl.BlockSpec((1, 1, 1, _CHUNK), lambda b, t: (b, t, 0, 0)),
            pl.BlockSpec((1, _NUM_EMBED), lambda b, t: (0, 0)),
            pl.BlockSpec((1, 128), lambda b, t: (0, 0)),
        ],
        out_shape=[
            jax.ShapeDtypeStruct((B, C, HW), jnp.float32),
            jax.ShapeDtypeStruct((B, n_chunks, 1, _CHUNK), jnp.int32),
            jax.ShapeDtypeStruct((1, _NUM_EMBED), jnp.float32),
            jax.ShapeDtypeStruct((1, 128), jnp.float32),
        ],
        compiler_params=pltpu.CompilerParams(
            dimension_semantics=("arbitrary", "arbitrary")),
    )(x3, emb_weight)

    loss = scalars[0, 0]
    perplexity = scalars[0, 1]
    q_out = q3.reshape(B, C, H, W)
    encoding_indices = idx3.reshape(B, H, W)
    return loss, q_out, perplexity, encoding_indices
